# baseline (speedup 1.0000x reference)
import functools
import math

import jax
import jax.numpy as jnp
from jax import lax
from jax.experimental import pallas as pl
from jax.experimental.pallas import tpu as pltpu
from jax.experimental.pallas import tpu_sc as plsc

D_MODEL = 512
SCALE = math.sqrt(float(D_MODEL))
LANES = 16

NUM_CORES = 2
NUM_SUBCORES = 16
NUM_WORKERS = NUM_CORES * NUM_SUBCORES

CHUNK = 64


def _make_emb_kernel(num_idx):
    assert num_idx % (NUM_WORKERS * CHUNK) == 0
    b_per_w = num_idx // NUM_WORKERS
    n_chunks = b_per_w // CHUNK
    vecs_per_row = D_MODEL // LANES

    mesh = plsc.VectorSubcoreMesh(core_axis_name="c", subcore_axis_name="s")

    @functools.partial(
        pl.kernel,
        out_type=jax.ShapeDtypeStruct((num_idx, D_MODEL), jnp.float32),
        mesh=mesh,
        scratch_types=[
            pltpu.VMEM((b_per_w,), jnp.int32),
            pltpu.VMEM((CHUNK, D_MODEL), jnp.float32),
            pltpu.SemaphoreType.DMA,
        ],
    )
    def emb(table_hbm, idx_hbm, out_hbm, idx_v, rows_v, sem):
        wid = lax.axis_index("s") * NUM_CORES + lax.axis_index("c")
        base = wid * b_per_w
        pltpu.sync_copy(idx_hbm.at[pl.ds(base, b_per_w)], idx_v)

        def chunk_body(c, carry):
            off = c * CHUNK
            pltpu.async_copy(
                table_hbm.at[idx_v.at[pl.ds(off, CHUNK)]], rows_v, sem
            ).wait()

            def scale_row(r, carry2):
                for j in range(vecs_per_row):
                    sl = pl.ds(j * LANES, LANES)
                    rows_v[r, sl] = rows_v[r, sl] * SCALE
                return carry2

            lax.fori_loop(0, CHUNK, scale_row, 0, unroll=False)
            pltpu.sync_copy(rows_v, out_hbm.at[pl.ds(base + off, CHUNK)])
            return carry

        lax.fori_loop(0, n_chunks, chunk_body, 0, unroll=False)

    return emb


@jax.jit
def kernel(x, table):
    b, s = x.shape
    idx = x.reshape(-1).astype(jnp.int32)
    emb = _make_emb_kernel(idx.shape[0])
    out = emb(table, idx)
    return out.reshape(b, s, D_MODEL)


# speedup vs baseline: 1.1787x; 1.1787x over previous
import functools
import math

import jax
import jax.numpy as jnp
from jax import lax
from jax.experimental import pallas as pl
from jax.experimental.pallas import tpu as pltpu
from jax.experimental.pallas import tpu_sc as plsc

D_MODEL = 512
SCALE = math.sqrt(float(D_MODEL))
LANES = 16

NUM_CORES = 2
NUM_SUBCORES = 16
NUM_WORKERS = NUM_CORES * NUM_SUBCORES

CHUNK = 40
NBUF = 2
VECS_PER_ROW = D_MODEL // LANES


def _make_emb_kernel(num_idx):
    assert num_idx % (NUM_WORKERS * CHUNK * NBUF) == 0
    b_per_w = num_idx // NUM_WORKERS
    n_chunks = b_per_w // CHUNK
    n_outer = n_chunks // NBUF

    mesh = plsc.VectorSubcoreMesh(core_axis_name="c", subcore_axis_name="s")

    @functools.partial(
        pl.kernel,
        out_type=jax.ShapeDtypeStruct((num_idx, D_MODEL), jnp.float32),
        mesh=mesh,
        scratch_types=[
            pltpu.VMEM((b_per_w,), jnp.int32),
            [pltpu.VMEM((CHUNK, D_MODEL), jnp.float32) for _ in range(NBUF)],
            [pltpu.VMEM((CHUNK, D_MODEL), jnp.float32) for _ in range(NBUF)],
            [pltpu.SemaphoreType.DMA for _ in range(NBUF)],
            [pltpu.SemaphoreType.DMA for _ in range(NBUF)],
        ],
    )
    def emb(table_hbm, idx_hbm, out_hbm, idx_v, gbufs, obufs, gsems, osems):
        wid = lax.axis_index("s") * NUM_CORES + lax.axis_index("c")
        base = wid * b_per_w
        pltpu.sync_copy(idx_hbm.at[pl.ds(base, b_per_w)], idx_v)

        def start_gather(c, b):
            pltpu.async_copy(
                table_hbm.at[idx_v.at[pl.ds(c * CHUNK, CHUNK)]],
                gbufs[b],
                gsems[b],
            )

        def gather_wait(b):
            pltpu.make_async_copy(
                table_hbm.at[idx_v.at[pl.ds(0, CHUNK)]], gbufs[b], gsems[b]
            ).wait()

        def out_descr(c, b):
            return pltpu.make_async_copy(
                obufs[b], out_hbm.at[pl.ds(base + c * CHUNK, CHUNK)], osems[b]
            )

        def scale(b):
            def scale_row(r, carry):
                for j in range(VECS_PER_ROW):
                    sl = pl.ds(j * LANES, LANES)
                    obufs[b][r, sl] = gbufs[b][r, sl] * SCALE
                return carry

            lax.fori_loop(0, CHUNK, scale_row, 0, unroll=False)

        for b in range(NBUF):
            start_gather(b, b)

        for b in range(NBUF):
            gather_wait(b)
            scale(b)
            out_descr(b, b).start()
            start_gather(NBUF + b, b)

        @pl.loop(1, n_outer - 1)
        def outer(o):
            c0 = o * NBUF
            for b in range(NBUF):
                c = c0 + b
                gather_wait(b)
                out_descr(c - NBUF, b).wait()
                scale(b)
                out_descr(c, b).start()
                start_gather(c + NBUF, b)

        for b in range(NBUF):
            c = (n_outer - 1) * NBUF + b
            gather_wait(b)
            out_descr(c - NBUF, b).wait()
            scale(b)
            out_descr(c, b).start()

        for b in range(NBUF):
            c = (n_outer - 1) * NBUF + b
            out_descr(c, b).wait()

    return emb


@jax.jit
def kernel(x, table):
    b, s = x.shape
    idx = x.reshape(-1).astype(jnp.int32)
    emb = _make_emb_kernel(idx.shape[0])
    out = emb(table, idx)
    return out.reshape(b, s, D_MODEL)
